# baseline (speedup 1.0000x reference)
import functools

import jax
import jax.numpy as jnp
from jax import lax
from jax.experimental import pallas as pl
from jax.experimental.pallas import tpu as pltpu
from jax.experimental.pallas import tpu_sc as plsc

B = 16384
D = 64
NEG_VOCAB = 256
S = 5
BLK = 2048


@functools.cache
def _make_sc_gather(V, d, b):
    info = plsc.get_sparse_core_info()
    nw = info.num_cores * info.num_subcores
    b_per_w = b // nw
    assert b % (8 * nw) == 0 and d % info.num_lanes == 0
    mesh = plsc.VectorSubcoreMesh(core_axis_name="c", subcore_axis_name="s")

    @functools.partial(
        pl.kernel,
        mesh=mesh,
        out_type=jax.ShapeDtypeStruct((b, d), jnp.float32),
        scratch_types=[
            pltpu.VMEM((b_per_w,), jnp.int32),
            pltpu.VMEM((b_per_w, d), jnp.float32),
            pltpu.SemaphoreType.DMA,
        ],
        compiler_params=pltpu.CompilerParams(use_tc_tiling_on_sc=False),
    )
    def gather_k(table_hbm, idx_hbm, out_hbm, idx_v, rows_v, sem):
        wid = lax.axis_index("s") * info.num_cores + lax.axis_index("c")
        base = wid * b_per_w
        pltpu.sync_copy(idx_hbm.at[pl.ds(base, b_per_w)], idx_v)
        pltpu.async_copy(table_hbm.at[idx_v], rows_v, sem).wait()
        pltpu.sync_copy(rows_v, out_hbm.at[pl.ds(base, b_per_w)])

    return gather_k


def _neg_body(h_ref, neg_ref, w256_ref, out_ref):
    i = pl.program_id(0)
    h = h_ref[...]
    neg = neg_ref[...]

    z_all = lax.dot_general(
        h, w256_ref[...], (((1,), (1,)), ((), ())),
        preferred_element_type=jnp.float32,
    )
    col = lax.broadcasted_iota(jnp.int32, (BLK, NEG_VOCAB), 1)
    neg_sum = jnp.float32(0.0)
    for s in range(S):
        m = col == neg[:, s:s + 1]
        z_s = jnp.sum(jnp.where(m, z_all, 0.0), axis=1)
        p_s = jax.nn.sigmoid(z_s)
        neg_sum += -jnp.sum(jnp.maximum(jnp.log(1.0 - p_s), -100.0))

    @pl.when(i == 0)
    def _():
        out_ref[...] = jnp.zeros_like(out_ref)

    out_ref[...] = out_ref[...] + neg_sum * (0.5 / (B * S))


def _pos_body(h_ref, wp_ref, negsum_ref, out_ref):
    i = pl.program_id(0)
    h = h_ref[...]
    wp = wp_ref[...]
    z_pos = jnp.sum(h * wp, axis=1)
    p_pos = jax.nn.sigmoid(z_pos)
    pos_sum = -jnp.sum(jnp.maximum(jnp.log(p_pos), -100.0))

    @pl.when(i == 0)
    def _():
        out_ref[...] = negsum_ref[...]

    out_ref[...] = out_ref[...] + pos_sum * (0.5 / B)


def _tc_neg(h, neg, w256):
    return pl.pallas_call(
        _neg_body,
        grid=(B // BLK,),
        in_specs=[
            pl.BlockSpec((BLK, D), lambda i: (i, 0)),
            pl.BlockSpec((BLK, S), lambda i: (i, 0)),
            pl.BlockSpec((NEG_VOCAB, D), lambda i: (0, 0)),
        ],
        out_specs=pl.BlockSpec((1, 1), lambda i: (0, 0)),
        out_shape=jax.ShapeDtypeStruct((1, 1), jnp.float32),
        compiler_params=pltpu.CompilerParams(
            dimension_semantics=("arbitrary",),
        ),
    )(h, neg, w256)


def _tc_pos(h, w_pos, neg_sum):
    out = pl.pallas_call(
        _pos_body,
        grid=(B // BLK,),
        in_specs=[
            pl.BlockSpec((BLK, D), lambda i: (i, 0)),
            pl.BlockSpec((BLK, D), lambda i: (i, 0)),
            pl.BlockSpec((1, 1), lambda i: (0, 0)),
        ],
        out_specs=pl.BlockSpec((1, 1), lambda i: (0, 0)),
        out_shape=jax.ShapeDtypeStruct((1, 1), jnp.float32),
        compiler_params=pltpu.CompilerParams(
            dimension_semantics=("arbitrary",),
        ),
    )(h, w_pos, neg_sum)
    return out[0, 0]


def kernel(h, target_index, negative_sample, W):
    idx = target_index.astype(jnp.int32)
    neg = negative_sample.astype(jnp.int32)
    w_pos = _make_sc_gather(W.shape[0], D, B)(W, idx)
    neg_sum = _tc_neg(h, neg, W[:NEG_VOCAB])
    return _tc_pos(h, w_pos, neg_sum)


# speedup vs baseline: 2.1410x; 2.1410x over previous
import functools

import jax
import jax.numpy as jnp
from jax import lax
from jax.experimental import pallas as pl
from jax.experimental.pallas import tpu as pltpu
from jax.experimental.pallas import tpu_sc as plsc

B = 16384
D = 64
NEG_VOCAB = 256
S = 5
BLK = 2048
NBLK = B // BLK
VROWS = 100000
WCH = 12800
P2 = 2 * D
V2 = NBLK * (WCH // 2)


def _eye64():
    r = lax.broadcasted_iota(jnp.int32, (D, D), 0)
    c = lax.broadcasted_iota(jnp.int32, (D, D), 1)
    return (r == c).astype(jnp.float32)


@functools.cache
def _make_sc_gather_packed(vp, dp, b):
    info = plsc.get_sparse_core_info()
    nw = info.num_cores * info.num_subcores
    b_per_w = b // nw
    assert b % (8 * nw) == 0 and dp % info.num_lanes == 0
    mesh = plsc.VectorSubcoreMesh(core_axis_name="c", subcore_axis_name="s")

    @functools.partial(
        pl.kernel,
        mesh=mesh,
        out_type=jax.ShapeDtypeStruct((b, dp), jnp.float32),
        scratch_types=[
            pltpu.VMEM((b_per_w,), jnp.int32),
            pltpu.VMEM((b_per_w, dp), jnp.float32),
            pltpu.SemaphoreType.DMA,
        ],
    )
    def gather_k(table_hbm, idx_hbm, out_hbm, idx_v, rows_v, sem):
        wid = lax.axis_index("s") * info.num_cores + lax.axis_index("c")
        base = wid * b_per_w
        pltpu.sync_copy(idx_hbm.at[pl.ds(base, b_per_w)], idx_v)
        pltpu.async_copy(table_hbm.at[idx_v], rows_v, sem).wait()
        pltpu.sync_copy(rows_v, out_hbm.at[pl.ds(base, b_per_w)])

    return gather_k


def _neg_repack_body(ht_ref, negt_ref, w256t_ref, wt_ref, out_ref, w2_ref):
    i = pl.program_id(0)

    eye = _eye64()
    half_a = lax.dot_general(
        wt_ref[:, :WCH // 2], eye, (((0,), (0,)), ((), ())),
        preferred_element_type=jnp.float32,
    )
    half_b = lax.dot_general(
        wt_ref[:, WCH // 2:], eye, (((0,), (0,)), ((), ())),
        preferred_element_type=jnp.float32,
    )
    w2_ref[...] = jnp.concatenate([half_a, half_b], axis=1)

    ht = ht_ref[...]
    negt = negt_ref[...]

    z_all = lax.dot_general(
        w256t_ref[...], ht, (((0,), (0,)), ((), ())),
        preferred_element_type=jnp.float32,
    )
    row = lax.broadcasted_iota(jnp.int32, (NEG_VOCAB, BLK), 0)
    neg_sum = jnp.float32(0.0)
    for s in range(S):
        m = row == negt[s][None, :]
        z_s = jnp.sum(jnp.where(m, z_all, 0.0), axis=0)
        p_s = jax.nn.sigmoid(z_s)
        neg_sum += -jnp.sum(jnp.maximum(jnp.log(1.0 - p_s), -100.0))

    @pl.when(i == 0)
    def _():
        out_ref[...] = jnp.zeros_like(out_ref)

    out_ref[...] = out_ref[...] + neg_sum * (0.5 / (B * S))


def _pos_body(ht_ref, wpp_ref, par_ref, negsum_ref, out_ref):
    i = pl.program_id(0)
    ht = ht_ref[...]
    wpp = wpp_ref[...]
    half = par_ref[...]
    r = lax.broadcasted_iota(jnp.int32, (P2, P2), 0)
    c = lax.broadcasted_iota(jnp.int32, (P2, P2), 1)
    eye2 = (r == c).astype(jnp.float32)
    wppt = lax.dot_general(
        eye2, wpp, (((1,), (1,)), ((), ())),
        preferred_element_type=jnp.float32,
    )
    wpt = jnp.where(half == 1, wppt[D:, :], wppt[:D, :])
    z_pos = jnp.sum(ht * wpt, axis=0)
    p_pos = jax.nn.sigmoid(z_pos)
    pos_sum = -jnp.sum(jnp.maximum(jnp.log(p_pos), -100.0))

    @pl.when(i == 0)
    def _():
        out_ref[...] = negsum_ref[...]

    out_ref[...] = out_ref[...] + pos_sum * (0.5 / B)


def _tc_neg_repack(ht, negt, w256t, wt):
    return pl.pallas_call(
        _neg_repack_body,
        grid=(NBLK,),
        in_specs=[
            pl.BlockSpec((D, BLK), lambda i: (0, i)),
            pl.BlockSpec((S, BLK), lambda i: (0, i)),
            pl.BlockSpec((D, NEG_VOCAB), lambda i: (0, 0)),
            pl.BlockSpec((D, WCH), lambda i: (0, i)),
        ],
        out_specs=[
            pl.BlockSpec((1, 1), lambda i: (0, 0)),
            pl.BlockSpec((WCH // 2, P2), lambda i: (i, 0)),
        ],
        out_shape=[
            jax.ShapeDtypeStruct((1, 1), jnp.float32),
            jax.ShapeDtypeStruct((V2, P2), jnp.float32),
        ],
        compiler_params=pltpu.CompilerParams(
            dimension_semantics=("arbitrary",),
        ),
    )(ht, negt, w256t, wt)


def _tc_pos(ht, wpp, par, neg_sum):
    out = pl.pallas_call(
        _pos_body,
        grid=(NBLK,),
        in_specs=[
            pl.BlockSpec((D, BLK), lambda i: (0, i)),
            pl.BlockSpec((BLK, P2), lambda i: (i, 0)),
            pl.BlockSpec((1, BLK), lambda i: (0, i)),
            pl.BlockSpec((1, 1), lambda i: (0, 0)),
        ],
        out_specs=pl.BlockSpec((1, 1), lambda i: (0, 0)),
        out_shape=jax.ShapeDtypeStruct((1, 1), jnp.float32),
        compiler_params=pltpu.CompilerParams(
            dimension_semantics=("arbitrary",),
        ),
    )(ht, wpp, par, neg_sum)
    return out[0, 0]


def kernel(h, target_index, negative_sample, W):
    idx = target_index.astype(jnp.int32)
    negt = jnp.transpose(negative_sample.astype(jnp.int32))
    ht = jnp.transpose(h)
    wt = jnp.transpose(W)
    w256t = wt[:, :NEG_VOCAB]
    neg_sum, w2 = _tc_neg_repack(ht, negt, w256t, wt)
    rem = idx % WCH
    prow = (idx // WCH) * (WCH // 2) + (rem % (WCH // 2))
    half = rem // (WCH // 2)
    wpp = _make_sc_gather_packed(V2, P2, B)(w2, prow)
    return _tc_pos(ht, wpp, jnp.reshape(half, (1, B)), neg_sum)
